# baseline (speedup 1.0000x reference)
import jax
import jax.numpy as jnp
from jax.experimental import pallas as pl
from jax.experimental.pallas import tpu as pltpu

_BR = 200
_NBUF = 5


def _gcn_kernel(x_ref, w_ref, b_ref, adj_ref, out_ref, h_ref, bufs, sems):
    n_chunks = adj_ref.shape[0] // _BR

    def _start(i):
        pltpu.make_async_copy(
            adj_ref.at[pl.ds(i * _BR, _BR), :],
            bufs.at[i % _NBUF],
            sems.at[i % _NBUF],
        ).start()

    for i in range(min(_NBUF, n_chunks)):
        _start(i)

    h = jax.lax.dot_general(
        x_ref[...], w_ref[...],
        (((1,), (1,)), ((), ())),
        preferred_element_type=jnp.float32,
    ) + b_ref[...]
    h_ref[...] = h.astype(jnp.bfloat16)
    hb = h_ref[...]

    def _body(i, carry):
        slot = i % _NBUF
        pltpu.make_async_copy(
            adj_ref.at[pl.ds(i * _BR, _BR), :],
            bufs.at[slot],
            sems.at[slot],
        ).wait()
        blk = bufs[slot].astype(jnp.bfloat16)

        @pl.when(i + _NBUF < n_chunks)
        def _():
            _start(i + _NBUF)

        out_ref[pl.ds(i * _BR, _BR), :] = jnp.maximum(
            jnp.dot(blk, hb, preferred_element_type=jnp.float32), 0.0)
        return carry

    jax.lax.fori_loop(0, n_chunks, _body, 0, unroll=False)


def kernel(x, adj, W, b):
    N, din = x.shape
    dout = W.shape[0]
    return pl.pallas_call(
        _gcn_kernel,
        in_specs=[
            pl.BlockSpec((N, din), lambda: (0, 0)),
            pl.BlockSpec((dout, din), lambda: (0, 0)),
            pl.BlockSpec((1, dout), lambda: (0, 0)),
            pl.BlockSpec(memory_space=pl.ANY),
        ],
        out_specs=pl.BlockSpec((N, dout), lambda: (0, 0)),
        out_shape=jax.ShapeDtypeStruct((N, dout), jnp.float32),
        scratch_shapes=[
            pltpu.VMEM((N, dout), jnp.bfloat16),
            pltpu.VMEM((_NBUF, _BR, N), jnp.float32),
            pltpu.SemaphoreType.DMA((_NBUF,)),
        ],
    )(x, W, b.reshape(1, dout), adj)


# speedup vs baseline: 1.0331x; 1.0331x over previous
import jax
import jax.numpy as jnp
from jax.experimental import pallas as pl
from jax.experimental.pallas import tpu as pltpu

_BM = 400


def _gcn_kernel(x_ref, w_ref, b_ref, adj_ref, out_ref, h_ref):
    @pl.when(pl.program_id(0) == 0)
    def _():
        h = jax.lax.dot_general(
            x_ref[...], w_ref[...],
            (((1,), (1,)), ((), ())),
            preferred_element_type=jnp.float32,
        ) + b_ref[...]
        h_ref[...] = h

    out_ref[...] = jnp.maximum(
        jax.lax.dot_general(
            adj_ref[...], h_ref[...],
            (((1,), (0,)), ((), ())),
            precision=jax.lax.Precision.DEFAULT,
            preferred_element_type=jnp.float32,
        ),
        0.0,
    )


def kernel(x, adj, W, b):
    N, din = x.shape
    dout = W.shape[0]
    return pl.pallas_call(
        _gcn_kernel,
        grid=(N // _BM,),
        in_specs=[
            pl.BlockSpec((N, din), lambda i: (0, 0)),
            pl.BlockSpec((dout, din), lambda i: (0, 0)),
            pl.BlockSpec((1, dout), lambda i: (0, 0)),
            pl.BlockSpec((_BM, N), lambda i: (i, 0)),
        ],
        out_specs=pl.BlockSpec((_BM, dout), lambda i: (i, 0)),
        out_shape=jax.ShapeDtypeStruct((N, dout), jnp.float32),
        scratch_shapes=[pltpu.VMEM((N, dout), jnp.float32)],
        compiler_params=pltpu.CompilerParams(
            dimension_semantics=("arbitrary",),
        ),
    )(x, W, b.reshape(1, dout), adj)


# speedup vs baseline: 1.0380x; 1.0048x over previous
import jax
import jax.numpy as jnp
from jax.experimental import pallas as pl
from jax.experimental.pallas import tpu as pltpu

_BM = 400
_KA = 256


def _gcn_kernel(x_ref, w_ref, b_ref, adj_ref, out_ref, xa_ref, g_ref):
    din = x_ref.shape[1]
    dout = w_ref.shape[0]

    @pl.when(pl.program_id(0) == 0)
    def _():
        xa_ref[:, :din] = x_ref[...].astype(jnp.bfloat16)
        n = x_ref.shape[0]
        ones_col = (jax.lax.broadcasted_iota(jnp.int32, (n, _KA - din), 1)
                    == 0).astype(jnp.bfloat16)
        xa_ref[:, din:] = ones_col
        g_ref[:din, :] = w_ref[...].T.astype(jnp.bfloat16)
        brow = (jax.lax.broadcasted_iota(jnp.int32, (_KA - din, dout), 0)
                == 0).astype(jnp.float32) * b_ref[...]
        g_ref[din:, :] = brow.astype(jnp.bfloat16)

    t = jnp.dot(adj_ref[...].astype(jnp.bfloat16), xa_ref[...],
                preferred_element_type=jnp.float32)
    out_ref[...] = jnp.maximum(
        jnp.dot(t.astype(jnp.bfloat16), g_ref[...],
                preferred_element_type=jnp.float32),
        0.0,
    )


def kernel(x, adj, W, b):
    N, din = x.shape
    dout = W.shape[0]
    return pl.pallas_call(
        _gcn_kernel,
        grid=(N // _BM,),
        in_specs=[
            pl.BlockSpec((N, din), lambda i: (0, 0)),
            pl.BlockSpec((dout, din), lambda i: (0, 0)),
            pl.BlockSpec((1, dout), lambda i: (0, 0)),
            pl.BlockSpec((_BM, N), lambda i: (i, 0)),
        ],
        out_specs=pl.BlockSpec((_BM, dout), lambda i: (i, 0)),
        out_shape=jax.ShapeDtypeStruct((N, dout), jnp.float32),
        scratch_shapes=[
            pltpu.VMEM((N, _KA), jnp.bfloat16),
            pltpu.VMEM((_KA, dout), jnp.bfloat16),
        ],
        compiler_params=pltpu.CompilerParams(
            dimension_semantics=("arbitrary",),
        ),
    )(x, W, b.reshape(1, dout), adj)


# speedup vs baseline: 1.0408x; 1.0026x over previous
import jax
import jax.numpy as jnp
from jax.experimental import pallas as pl
from jax.experimental.pallas import tpu as pltpu

_BM = 400


def _gcn_kernel(x_ref, w_ref, b_ref, adj_ref, out_ref, h_ref):
    @pl.when(pl.program_id(0) == 0)
    def _():
        h = jax.lax.dot_general(
            x_ref[...], w_ref[...],
            (((1,), (1,)), ((), ())),
            preferred_element_type=jnp.float32,
        ) + b_ref[...]
        h_ref[...] = h.astype(jnp.bfloat16)

    out_ref[...] = jnp.maximum(
        jnp.dot(adj_ref[...].astype(jnp.bfloat16), h_ref[...],
                preferred_element_type=jnp.float32),
        0.0,
    )


def kernel(x, adj, W, b):
    N, din = x.shape
    dout = W.shape[0]
    return pl.pallas_call(
        _gcn_kernel,
        grid=(N // _BM,),
        in_specs=[
            pl.BlockSpec((N, din), lambda i: (0, 0)),
            pl.BlockSpec((dout, din), lambda i: (0, 0)),
            pl.BlockSpec((1, dout), lambda i: (0, 0)),
            pl.BlockSpec((_BM, N), lambda i: (i, 0)),
        ],
        out_specs=pl.BlockSpec((_BM, dout), lambda i: (i, 0)),
        out_shape=jax.ShapeDtypeStruct((N, dout), jnp.float32),
        scratch_shapes=[pltpu.VMEM((N, dout), jnp.bfloat16)],
        compiler_params=pltpu.CompilerParams(
            dimension_semantics=("arbitrary",),
        ),
    )(x, W, b.reshape(1, dout), adj)
